# Initial kernel scaffold; baseline (speedup 1.0000x reference)
import jax
import jax.numpy as jnp
from jax.experimental import pallas as pl


def kernel(n_feats, e_feats, edge_index, W1e, b1e, W2e, b2e, W3e, b3e, W1n, b1n, W2n, b2n, W3n, b3n):
    raise NotImplementedError("write your pallas kernel here")


# speedup vs baseline: 3.3912x; 3.3912x over previous
import functools

import jax
import jax.numpy as jnp
from jax import lax
from jax.experimental import pallas as pl
from jax.experimental.pallas import tpu as pltpu
from jax.experimental.pallas import tpu_sc as plsc

NC = 2
NS = 16
NW = NC * NS
GRP = 128
GPC = 4
CHUNK = GRP * GPC


def _edge_pre(e_feats, W1e_e, b1e, blk):
    E, EIN = e_feats.shape
    LAT = W1e_e.shape[1]

    def body(e_ref, w_ref, b_ref, o_ref):
        o_ref[...] = (
            jnp.dot(e_ref[...], w_ref[...], preferred_element_type=jnp.float32)
            + b_ref[...]
        )

    return pl.pallas_call(
        body,
        grid=(E // blk,),
        in_specs=[
            pl.BlockSpec((blk, EIN), lambda i: (i, 0)),
            pl.BlockSpec((EIN, LAT), lambda i: (0, 0)),
            pl.BlockSpec((1, LAT), lambda i: (0, 0)),
        ],
        out_specs=pl.BlockSpec((blk, LAT), lambda i: (i, 0)),
        out_shape=jax.ShapeDtypeStruct((E, LAT), jnp.float32),
    )(e_feats, W1e_e, b1e.reshape(1, LAT))


def _node_pre(n_feats, W1e_s, W1e_d, blk):
    N, DIN = n_feats.shape
    LAT = W1e_s.shape[1]

    def body(n_ref, ws_ref, wd_ref, p_ref, q_ref):
        x = n_ref[...]
        p_ref[...] = jnp.dot(x, ws_ref[...], preferred_element_type=jnp.float32)
        q_ref[...] = jnp.dot(x, wd_ref[...], preferred_element_type=jnp.float32)

    return pl.pallas_call(
        body,
        grid=(N // blk,),
        in_specs=[
            pl.BlockSpec((blk, DIN), lambda i: (i, 0)),
            pl.BlockSpec((DIN, LAT), lambda i: (0, 0)),
            pl.BlockSpec((DIN, LAT), lambda i: (0, 0)),
        ],
        out_specs=[
            pl.BlockSpec((blk, LAT), lambda i: (i, 0)),
            pl.BlockSpec((blk, LAT), lambda i: (i, 0)),
        ],
        out_shape=[
            jax.ShapeDtypeStruct((N, LAT), jnp.float32),
            jax.ShapeDtypeStruct((N, LAT), jnp.float32),
        ],
    )(n_feats, W1e_s, W1e_d)


def _sc_gather_add(src_r, dst_r, ef, P, Q):
    E, LAT = ef.shape
    n_chunks = E // CHUNK
    iters = -(-n_chunks // NW)
    mesh = plsc.VectorSubcoreMesh(core_axis_name="c", subcore_axis_name="s")

    @functools.partial(
        pl.kernel,
        out_type=jax.ShapeDtypeStruct((E, LAT), jnp.float32),
        mesh=mesh,
        scratch_types=[
            pltpu.VMEM((GPC, GRP), jnp.int32),
            pltpu.VMEM((GPC, GRP), jnp.int32),
            pltpu.VMEM((CHUNK, LAT), jnp.float32),
            pltpu.VMEM((CHUNK, LAT), jnp.float32),
            pltpu.VMEM((CHUNK, LAT), jnp.float32),
            pltpu.SemaphoreType.DMA,
        ],
        compiler_params=pltpu.CompilerParams(use_tc_tiling_on_sc=False),
    )
    def k(src_hbm, dst_hbm, ef_hbm, p_hbm, q_hbm, h1_hbm, isrc, idst, efb, pb, qb, sem):
        wid = lax.axis_index("s") * NC + lax.axis_index("c")

        def body(i, carry):
            cid = wid + i * NW

            @pl.when(cid < n_chunks)
            def _():
                base = cid * CHUNK
                g0 = cid * GPC
                ci = pltpu.async_copy(src_hbm.at[pl.ds(g0, GPC)], isrc, sem)
                cj = pltpu.async_copy(dst_hbm.at[pl.ds(g0, GPC)], idst, sem)
                ce = pltpu.async_copy(ef_hbm.at[pl.ds(base, CHUNK)], efb, sem)
                ci.wait()
                cj.wait()
                gs = []
                for g in range(GPC):
                    sl = pl.ds(g * GRP, GRP)
                    gs.append(pltpu.async_copy(p_hbm.at[isrc.at[g]], pb.at[sl], sem))
                    gs.append(pltpu.async_copy(q_hbm.at[idst.at[g]], qb.at[sl], sem))
                ce.wait()
                for gcopy in gs:
                    gcopy.wait()

                def rows(j, c2):
                    for u in range(8):
                        r = j * 8 + u
                        efb[r] = jnp.maximum(efb[r] + pb[r] + qb[r], 0.0)
                    return c2

                lax.fori_loop(0, CHUNK // 8, rows, 0)
                pltpu.sync_copy(efb, h1_hbm.at[pl.ds(base, CHUNK)])

            return carry

        lax.fori_loop(0, iters, body, 0)

    return k(src_r, dst_r, ef, P, Q)


def _edge_tail(h1, W2e, b2e, W3e, b3e, blk):
    E, LAT = h1.shape
    DOUT = W3e.shape[1]

    def body(h_ref, w2, b2, w3, b3, o_ref):
        h = jnp.maximum(
            jnp.dot(h_ref[...], w2[...], preferred_element_type=jnp.float32) + b2[...],
            0.0,
        )
        z = jnp.dot(h, w3[...], preferred_element_type=jnp.float32) + b3[...]
        o_ref[...] = jax.nn.sigmoid(z)

    return pl.pallas_call(
        body,
        grid=(E // blk,),
        in_specs=[
            pl.BlockSpec((blk, LAT), lambda i: (i, 0)),
            pl.BlockSpec((LAT, LAT), lambda i: (0, 0)),
            pl.BlockSpec((1, LAT), lambda i: (0, 0)),
            pl.BlockSpec((LAT, DOUT), lambda i: (0, 0)),
            pl.BlockSpec((1, DOUT), lambda i: (0, 0)),
        ],
        out_specs=pl.BlockSpec((blk, DOUT), lambda i: (i, 0)),
        out_shape=jax.ShapeDtypeStruct((E, DOUT), jnp.float32),
    )(h1, W2e, b2e.reshape(1, LAT), W3e, b3e.reshape(1, DOUT))


def _sc_segment_sum(dst_r, e_out, N, LAT):
    E, DOUT = e_out.shape
    HALF = DOUT // NC
    n_chunks = E // CHUNK
    rs = (N // NS) // 8 * 8
    rs_last = N - (NS - 1) * rs
    ZR = 16
    nz = N // ZR
    mesh = plsc.VectorSubcoreMesh(core_axis_name="c", subcore_axis_name="s")

    @functools.partial(
        pl.kernel,
        out_type=[
            jax.ShapeDtypeStruct((N, DOUT), jnp.float32),
            jax.ShapeDtypeStruct((N, LAT), jnp.float32),
        ],
        mesh=mesh,
        scratch_types=[
            pltpu.VMEM((GPC, GRP), jnp.int32),
            pltpu.VMEM((CHUNK, HALF), jnp.float32),
            pltpu.VMEM((GRP, LAT), jnp.float32),
            pltpu.VMEM((ZR, HALF), jnp.float32),
            pltpu.VMEM((ZR, LAT), jnp.float32),
            pltpu.VMEM_SHARED((N, HALF), jnp.float32),
            pltpu.VMEM_SHARED((N, LAT), jnp.float32),
            pltpu.SemaphoreType.DMA,
        ],
        compiler_params=pltpu.CompilerParams(use_tc_tiling_on_sc=False),
    )
    def k(dst_hbm, eout_hbm, sums_hbm, deg_hbm,
          idxb, rows, onesb, zbs, zbd, ssh, dsh, sem):
        c = lax.axis_index("c")
        s = lax.axis_index("s")

        zero16 = jnp.zeros((16,), jnp.float32)
        one16 = jnp.ones((16,), jnp.float32)
        for r in range(ZR):
            for c8 in range(HALF // 16):
                zbs[r, pl.ds(c8 * 16, 16)] = zero16
            for c8 in range(LAT // 16):
                zbd[r, pl.ds(c8 * 16, 16)] = zero16
        for r in range(GRP):
            for c8 in range(LAT // 16):
                onesb[r, pl.ds(c8 * 16, 16)] = one16

        def zbody(i, carry):
            j = s + i * NS

            @pl.when(j < nz)
            def _():
                pltpu.sync_copy(zbs, ssh.at[pl.ds(j * ZR, ZR)])
                pltpu.sync_copy(zbd, dsh.at[pl.ds(j * ZR, ZR)])

            return carry

        lax.fori_loop(0, -(-nz // NS), zbody, 0)
        plsc.subcore_barrier()

        def body(i, carry):
            cid = s + i * NS

            @pl.when(cid < n_chunks)
            def _():
                base = cid * CHUNK
                g0 = cid * GPC
                ci = pltpu.async_copy(dst_hbm.at[pl.ds(g0, GPC)], idxb, sem)
                cr = pltpu.async_copy(
                    eout_hbm.at[pl.ds(base, CHUNK), pl.ds(c * HALF, HALF)],
                    rows, sem)
                ci.wait()
                cr.wait()
                for g in range(GPC):
                    sl = pl.ds(g * GRP, GRP)
                    pltpu.sync_copy(rows.at[sl], ssh.at[idxb.at[g]], add=True)

                @pl.when(c == 0)
                def _():
                    for g in range(GPC):
                        pltpu.sync_copy(onesb, dsh.at[idxb.at[g]], add=True)

            return carry

        lax.fori_loop(0, -(-n_chunks // NS), body, 0)
        plsc.subcore_barrier()

        @pl.when(s < NS - 1)
        def _():
            pltpu.sync_copy(ssh.at[pl.ds(s * rs, rs)],
                            sums_hbm.at[pl.ds(s * rs, rs), pl.ds(c * HALF, HALF)])

        @pl.when(s == NS - 1)
        def _():
            base = (NS - 1) * rs
            pltpu.sync_copy(ssh.at[pl.ds(base, rs_last)],
                            sums_hbm.at[pl.ds(base, rs_last), pl.ds(c * HALF, HALF)])

        @pl.when((c == 0) & (s < NS - 1))
        def _():
            pltpu.sync_copy(dsh.at[pl.ds(s * rs, rs)], deg_hbm.at[pl.ds(s * rs, rs)])

        @pl.when((c == 0) & (s == NS - 1))
        def _():
            base = (NS - 1) * rs
            pltpu.sync_copy(dsh.at[pl.ds(base, rs_last)], deg_hbm.at[pl.ds(base, rs_last)])

    return k(dst_r, e_out)


def _node_mlp(sums_p, deg_p, n_feats, W1n_a, W1n_b, b1n, W2n, b2n, W3n, b3n, blk):
    N, DIN = n_feats.shape
    LAT = W1n_a.shape[1]
    DOUT = W3n.shape[1]

    def body(sp, dp, nf, w1a, w1b, b1, w2, b2, w3, b3, o_ref):
        sums = sp[...]
        deg = dp[...]
        hN = sums / jnp.maximum(deg[:, :1], 1.0)
        h = jnp.maximum(
            jnp.dot(nf[...], w1a[...], preferred_element_type=jnp.float32)
            + jnp.dot(hN, w1b[...], preferred_element_type=jnp.float32)
            + b1[...],
            0.0,
        )
        h = jnp.maximum(
            jnp.dot(h, w2[...], preferred_element_type=jnp.float32) + b2[...], 0.0
        )
        o_ref[...] = jax.nn.sigmoid(
            jnp.dot(h, w3[...], preferred_element_type=jnp.float32) + b3[...]
        )

    DSUM = sums_p.shape[1]
    DDEG = deg_p.shape[1]
    return pl.pallas_call(
        body,
        grid=(N // blk,),
        in_specs=[
            pl.BlockSpec((blk, DSUM), lambda i: (i, 0)),
            pl.BlockSpec((blk, DDEG), lambda i: (i, 0)),
            pl.BlockSpec((blk, DIN), lambda i: (i, 0)),
            pl.BlockSpec((DIN, LAT), lambda i: (0, 0)),
            pl.BlockSpec((DSUM, LAT), lambda i: (0, 0)),
            pl.BlockSpec((1, LAT), lambda i: (0, 0)),
            pl.BlockSpec((LAT, LAT), lambda i: (0, 0)),
            pl.BlockSpec((1, LAT), lambda i: (0, 0)),
            pl.BlockSpec((LAT, DOUT), lambda i: (0, 0)),
            pl.BlockSpec((1, DOUT), lambda i: (0, 0)),
        ],
        out_specs=pl.BlockSpec((blk, DOUT), lambda i: (i, 0)),
        out_shape=jax.ShapeDtypeStruct((N, DOUT), jnp.float32),
    )(sums_p, deg_p, n_feats, W1n_a, W1n_b, b1n.reshape(1, LAT), W2n,
      b2n.reshape(1, LAT), W3n, b3n.reshape(1, DOUT))


def kernel(n_feats, e_feats, edge_index, W1e, b1e, W2e, b2e, W3e, b3e,
           W1n, b1n, W2n, b2n, W3n, b3n):
    N, DIN = n_feats.shape
    E, EIN = e_feats.shape
    LAT = W1e.shape[1]

    src_r = edge_index[0].reshape(E // GRP, GRP)
    dst_r = edge_index[1].reshape(E // GRP, GRP)
    W1e_e = W1e[:EIN]
    W1e_s = W1e[EIN:EIN + DIN]
    W1e_d = W1e[EIN + DIN:]
    W1n_a = W1n[:DIN]
    W1n_b = W1n[DIN:]

    ef = _edge_pre(e_feats, W1e_e, b1e, blk=1600)
    P, Q = _node_pre(n_feats, W1e_s, W1e_d, blk=1000)
    h1 = _sc_gather_add(src_r, dst_r, ef, P, Q)
    e_out = _edge_tail(h1, W2e, b2e, W3e, b3e, blk=1600)
    sums_p, deg_p = _sc_segment_sum(dst_r, e_out, N, LAT)
    n_out = _node_mlp(sums_p, deg_p, n_feats, W1n_a, W1n_b, b1n,
                      W2n, b2n, W3n, b3n, blk=1000)
    return (n_out, e_out)
